# baseline (device time: 861797 ns/iter reference)
import jax
import jax.numpy as jnp
from jax import lax
from jax.experimental import pallas as pl
from jax.experimental.pallas import tpu as pltpu

MESH = pl.DeviceIdType.MESH

_SIZES = [1024] * 15 + [512, 256, 128, 128]


def _schedule(half):
    assert sum(_SIZES) == half
    starts = []
    s = 0
    for sz in _SIZES:
        starts.append(s)
        s += sz
    return list(zip(starts, _SIZES))


def kernel(x):
    m, n = x.shape
    half = m // 2
    chunks = _schedule(half)
    nchunk = len(chunks)
    max_cr = max(sz for _, sz in chunks)

    def body(x_hbm, out_hbm, a_vmem, b_vmem, send1, recv1, send2, recv2,
             in_sems, out_sems, credit_sem):
        my_x = lax.axis_index("x")
        my_y = lax.axis_index("y")
        x_nbr = (1 - my_x, my_y)
        y_nbr = (my_x, 1 - my_y)

        barrier = pltpu.get_barrier_semaphore()
        for nbr in (x_nbr, y_nbr):
            pl.semaphore_signal(barrier, inc=1, device_id=nbr,
                                device_id_type=MESH)
        pl.semaphore_wait(barrier, 2)

        off = my_y * half
        stage = (1 - my_y) * half

        rdma1 = []
        for i, (c0r, cr) in enumerate(chunks):
            r = pltpu.make_async_remote_copy(
                src_ref=x_hbm.at[pl.ds(off + c0r, cr)],
                dst_ref=out_hbm.at[pl.ds(stage + c0r, cr)],
                send_sem=send1.at[i],
                recv_sem=recv1.at[i],
                device_id=x_nbr,
                device_id_type=MESH,
            )
            r.start()
            rdma1.append(r)

        rdma2 = []
        c2s = []
        for i, (c0r, cr) in enumerate(chunks):
            s = i % 2
            if i >= 2:
                rdma2[i - 2].wait_send()
                c2s[i - 2].wait()
            c0 = pltpu.make_async_copy(
                x_hbm.at[pl.ds(off + c0r, cr)],
                a_vmem.at[s, pl.ds(0, cr)], in_sems.at[s])
            c0.start()
            rdma1[i].wait_recv()
            c1 = pltpu.make_async_copy(
                out_hbm.at[pl.ds(stage + c0r, cr)],
                b_vmem.at[s, pl.ds(0, cr)], in_sems.at[2 + s])
            c1.start()
            c0.wait()
            c1.wait()
            pl.semaphore_signal(credit_sem, inc=1, device_id=y_nbr,
                                device_id_type=MESH)
            a_vmem[s, :cr] = a_vmem[s, :cr] + b_vmem[s, :cr]
            c2 = pltpu.make_async_copy(
                a_vmem.at[s, pl.ds(0, cr)], out_hbm.at[pl.ds(off + c0r, cr)],
                out_sems.at[s])
            c2.start()
            c2s.append(c2)
            pl.semaphore_wait(credit_sem, 1)
            r2 = pltpu.make_async_remote_copy(
                src_ref=a_vmem.at[s, pl.ds(0, cr)],
                dst_ref=out_hbm.at[pl.ds(off + c0r, cr)],
                send_sem=send2.at[i],
                recv_sem=recv2.at[i],
                device_id=y_nbr,
                device_id_type=MESH,
            )
            r2.start()
            rdma2.append(r2)

        for i in range(nchunk):
            rdma1[i].wait_send()
            rdma2[i].wait_recv()
        for i in (nchunk - 2, nchunk - 1):
            rdma2[i].wait_send()
            c2s[i].wait()

    return pl.pallas_call(
        body,
        out_shape=jax.ShapeDtypeStruct((m, n), x.dtype),
        in_specs=[pl.BlockSpec(memory_space=pl.ANY)],
        out_specs=pl.BlockSpec(memory_space=pl.ANY),
        scratch_shapes=[
            pltpu.VMEM((2, max_cr, n), x.dtype),
            pltpu.VMEM((2, max_cr, n), x.dtype),
            pltpu.SemaphoreType.DMA((nchunk,)),
            pltpu.SemaphoreType.DMA((nchunk,)),
            pltpu.SemaphoreType.DMA((nchunk,)),
            pltpu.SemaphoreType.DMA((nchunk,)),
            pltpu.SemaphoreType.DMA((4,)),
            pltpu.SemaphoreType.DMA((2,)),
            pltpu.SemaphoreType.REGULAR,
        ],
        compiler_params=pltpu.CompilerParams(collective_id=0),
    )(x)
